# baseline (speedup 1.0000x reference)
import functools

import jax
import jax.numpy as jnp
from jax import lax
from jax.experimental import pallas as pl
from jax.experimental.pallas import tpu as pltpu
from jax.experimental.pallas import tpu_sc as plsc

R = 1000
D = 128
B = 16384
NC = 2
NS = 16
NW = NC * NS
B_PER_W = B // NW
CHUNK = 64
NCHUNK = B_PER_W // CHUNK
HBM_CH = 2
ROWS_PER_STAGE = 64


def _make_gather():
  mesh = plsc.VectorSubcoreMesh(core_axis_name="c", subcore_axis_name="s")

  @functools.partial(
      pl.kernel,
      mesh=mesh,
      out_type=jax.ShapeDtypeStruct((B, D), jnp.float32),
      scratch_types=[
          pltpu.VMEM((B_PER_W,), jnp.int32),
          pltpu.VMEM((B_PER_W, D), jnp.float32),
          pltpu.VMEM_SHARED((R, D), jnp.float32),
          pltpu.SemaphoreType.DMA((NCHUNK,)),
          pltpu.SemaphoreType.DMA,
          pltpu.SemaphoreType.DMA,
      ],
  )
  def gather_kernel(idx_hbm, table_hbm, out_hbm, idx_v, rows_v, tab_sh,
                    gsems, wsem, ssem):
    sid = lax.axis_index("s")
    wid = sid * NC + lax.axis_index("c")
    base = wid * B_PER_W

    off = pl.multiple_of(jnp.minimum(sid * ROWS_PER_STAGE, R - ROWS_PER_STAGE),
                         8)
    stage = pltpu.async_copy(table_hbm.at[pl.ds(off, ROWS_PER_STAGE)],
                             tab_sh.at[pl.ds(off, ROWS_PER_STAGE)], ssem)
    pltpu.sync_copy(idx_hbm.at[pl.ds(base, B_PER_W)], idx_v)

    def fire_gather(src, j):
      return pltpu.async_copy(
          src.at[idx_v.at[pl.ds(j * CHUNK, CHUNK)]],
          rows_v.at[pl.ds(j * CHUNK, CHUNK)],
          gsems.at[j],
      )

    gathers = [fire_gather(table_hbm, j) for j in range(HBM_CH)]
    stage.wait()
    plsc.subcore_barrier()
    gathers += [fire_gather(tab_sh, j) for j in range(HBM_CH, NCHUNK)]

    writes = []
    for j in range(NCHUNK):
      gathers[j].wait()
      writes.append(
          pltpu.async_copy(
              rows_v.at[pl.ds(j * CHUNK, CHUNK)],
              out_hbm.at[pl.ds(base + j * CHUNK, CHUNK)],
              wsem,
          )
      )
    for c in writes:
      c.wait()

  return gather_kernel


_gather = _make_gather()


@jax.jit
def kernel(table, ks):
  return _gather(ks.astype(jnp.int32), table)


# speedup vs baseline: 1.0259x; 1.0259x over previous
import functools

import jax
import jax.numpy as jnp
from jax import lax
from jax.experimental import pallas as pl
from jax.experimental.pallas import tpu as pltpu
from jax.experimental.pallas import tpu_sc as plsc

R = 1000
D = 128
B = 16384
NC = 2
NS = 16
NW = NC * NS
B_PER_W = B // NW
CHUNK = 64
NCHUNK = B_PER_W // CHUNK
NSTAGE = 8
ROWS_PER_STAGE = 128
LAST_STAGE_ROWS = R - (NSTAGE - 1) * ROWS_PER_STAGE


def _make_gather():
  mesh = plsc.VectorSubcoreMesh(core_axis_name="c", subcore_axis_name="s")

  @functools.partial(
      pl.kernel,
      mesh=mesh,
      out_type=jax.ShapeDtypeStruct((B, D), jnp.float32),
      scratch_types=[
          pltpu.VMEM((B_PER_W,), jnp.int32),
          pltpu.VMEM((B_PER_W, D), jnp.float32),
          pltpu.VMEM_SHARED((R, D), jnp.float32),
          pltpu.SemaphoreType.DMA((NCHUNK,)),
          pltpu.SemaphoreType.DMA,
          pltpu.SemaphoreType.DMA,
      ],
  )
  def gather_kernel(idx_hbm, table_hbm, out_hbm, idx_v, rows_v, tab_sh,
                    gsems, wsem, ssem):
    sid = lax.axis_index("s")
    wid = sid * NC + lax.axis_index("c")
    base = wid * B_PER_W

    @pl.when(sid < NSTAGE - 1)
    def _stage():
      r0 = sid * ROWS_PER_STAGE
      stage = pltpu.async_copy(table_hbm.at[pl.ds(r0, ROWS_PER_STAGE)],
                               tab_sh.at[pl.ds(r0, ROWS_PER_STAGE)], ssem)
      pltpu.sync_copy(idx_hbm.at[pl.ds(base, B_PER_W)], idx_v)
      stage.wait()

    @pl.when(sid == NSTAGE - 1)
    def _stage_last():
      r0 = (NSTAGE - 1) * ROWS_PER_STAGE
      stage = pltpu.async_copy(table_hbm.at[pl.ds(r0, LAST_STAGE_ROWS)],
                               tab_sh.at[pl.ds(r0, LAST_STAGE_ROWS)], ssem)
      pltpu.sync_copy(idx_hbm.at[pl.ds(base, B_PER_W)], idx_v)
      stage.wait()

    @pl.when(sid >= NSTAGE)
    def _no_stage():
      pltpu.sync_copy(idx_hbm.at[pl.ds(base, B_PER_W)], idx_v)

    plsc.subcore_barrier()

    def fire_gather(j):
      return pltpu.async_copy(
          tab_sh.at[idx_v.at[pl.ds(j * CHUNK, CHUNK)]],
          rows_v.at[pl.ds(j * CHUNK, CHUNK)],
          gsems.at[j],
      )

    gathers = [fire_gather(j) for j in range(NCHUNK)]
    writes = []
    for j in range(NCHUNK):
      gathers[j].wait()
      writes.append(
          pltpu.async_copy(
              rows_v.at[pl.ds(j * CHUNK, CHUNK)],
              out_hbm.at[pl.ds(base + j * CHUNK, CHUNK)],
              wsem,
          )
      )
    for c in writes:
      c.wait()

  return gather_kernel


_gather = _make_gather()


@jax.jit
def kernel(table, ks):
  return _gather(ks.astype(jnp.int32), table)
